# baseline (speedup 1.0000x reference)
import functools

import jax
import jax.numpy as jnp
from jax import lax
from jax.experimental import pallas as pl
from jax.experimental.pallas import tpu as pltpu
from jax.experimental.pallas import tpu_sc as plsc

VOCAB = 100000
EMBED = 16
BATCH = 1024
TILE = 2048
NTILES = (VOCAB + TILE - 1) // TILE


def _gather_sc(table, idx):
    info = plsc.get_sparse_core_info()
    nc, ns = info.num_cores, info.num_subcores
    nw = nc * ns
    bpw = BATCH // nw
    mesh = plsc.VectorSubcoreMesh(core_axis_name="c", subcore_axis_name="s")

    @functools.partial(
        pl.kernel,
        mesh=mesh,
        compiler_params=pltpu.CompilerParams(use_tc_tiling_on_sc=False),
        out_type=jax.ShapeDtypeStruct((BATCH, EMBED), jnp.float32),
        scratch_types=[
            pltpu.VMEM((bpw,), jnp.int32),
            pltpu.VMEM((bpw, EMBED), jnp.float32),
            pltpu.SemaphoreType.DMA,
        ],
    )
    def gk(table_hbm, idx_hbm, out_hbm, idx_v, rows_v, sem):
        wid = lax.axis_index("s") * nc + lax.axis_index("c")
        base = wid * bpw
        pltpu.sync_copy(idx_hbm.at[pl.ds(base, bpw)], idx_v)
        pltpu.async_copy(table_hbm.at[idx_v], rows_v, sem).wait()
        pltpu.sync_copy(rows_v, out_hbm.at[pl.ds(base, bpw)])

    return gk(table, idx)


def _pass1_lse(h, wt, b2):
    def k(h_ref, w_ref, b_ref, lse_ref, m_ref, s_ref):
        j = pl.program_id(0)

        @pl.when(j == 0)
        def _():
            m_ref[...] = jnp.full((BATCH, 1), -1e30, jnp.float32)
            s_ref[...] = jnp.zeros((BATCH, 1), jnp.float32)

        logits = (
            jnp.dot(h_ref[...], w_ref[...], preferred_element_type=jnp.float32)
            + b_ref[...]
        )
        col = lax.broadcasted_iota(jnp.int32, (1, TILE), 1) + j * TILE
        logits = jnp.where(col < VOCAB, logits, -1e30)
        m_old = m_ref[...]
        m_new = jnp.maximum(m_old, jnp.max(logits, axis=1, keepdims=True))
        s_ref[...] = s_ref[...] * jnp.exp(m_old - m_new) + jnp.sum(
            jnp.exp(logits - m_new), axis=1, keepdims=True
        )
        m_ref[...] = m_new

        @pl.when(j == NTILES - 1)
        def _():
            lse_ref[...] = m_ref[...] + jnp.log(s_ref[...])

    return pl.pallas_call(
        k,
        grid=(NTILES,),
        in_specs=[
            pl.BlockSpec((BATCH, EMBED), lambda j: (0, 0)),
            pl.BlockSpec((EMBED, TILE), lambda j: (0, j)),
            pl.BlockSpec((1, TILE), lambda j: (0, j)),
        ],
        out_specs=pl.BlockSpec((BATCH, 1), lambda j: (0, 0)),
        out_shape=jax.ShapeDtypeStruct((BATCH, 1), jnp.float32),
        scratch_shapes=[
            pltpu.VMEM((BATCH, 1), jnp.float32),
            pltpu.VMEM((BATCH, 1), jnp.float32),
        ],
    )(h, wt, b2)


def _pass2_out(h, wt, b2, lse):
    def k(h_ref, w_ref, b_ref, lse_ref, o_ref):
        logits = jnp.dot(h_ref[...], w_ref[...], preferred_element_type=jnp.float32)
        o_ref[...] = logits + (b_ref[...] - lse_ref[...])

    return pl.pallas_call(
        k,
        grid=(NTILES,),
        in_specs=[
            pl.BlockSpec((BATCH, EMBED), lambda j: (0, 0)),
            pl.BlockSpec((EMBED, TILE), lambda j: (0, j)),
            pl.BlockSpec((1, TILE), lambda j: (0, j)),
            pl.BlockSpec((BATCH, 1), lambda j: (0, 0)),
        ],
        out_specs=pl.BlockSpec((BATCH, TILE), lambda j: (0, j)),
        out_shape=jax.ShapeDtypeStruct((BATCH, VOCAB), jnp.float32),
    )(h, wt, b2, lse)


def kernel(x, table, W, b):
    h = _gather_sc(table, x.astype(jnp.int32))
    wt = W.T
    b2 = b.reshape(1, VOCAB)
    lse = _pass1_lse(h, wt, b2)
    return _pass2_out(h, wt, b2, lse)


# speedup vs baseline: 1.0005x; 1.0005x over previous
import functools

import jax
import jax.numpy as jnp
from jax import lax
from jax.experimental import pallas as pl
from jax.experimental.pallas import tpu as pltpu
from jax.experimental.pallas import tpu_sc as plsc

VOCAB = 100000
EMBED = 16
BATCH = 1024
TILE = 2048
NTILES = (VOCAB + TILE - 1) // TILE


def _gather_sc(table, idx):
    info = plsc.get_sparse_core_info()
    nc, ns = info.num_cores, info.num_subcores
    nw = nc * ns
    bpw = BATCH // nw
    mesh = plsc.VectorSubcoreMesh(core_axis_name="c", subcore_axis_name="s")

    @functools.partial(
        pl.kernel,
        mesh=mesh,
        compiler_params=pltpu.CompilerParams(use_tc_tiling_on_sc=False),
        out_type=jax.ShapeDtypeStruct((BATCH, EMBED), jnp.float32),
        scratch_types=[
            pltpu.VMEM((bpw,), jnp.int32),
            pltpu.VMEM((bpw, EMBED), jnp.float32),
            pltpu.SemaphoreType.DMA,
        ],
    )
    def gk(table_hbm, idx_hbm, out_hbm, idx_v, rows_v, sem):
        wid = lax.axis_index("s") * nc + lax.axis_index("c")
        base = wid * bpw
        pltpu.sync_copy(idx_hbm.at[pl.ds(base, bpw)], idx_v)
        pltpu.async_copy(table_hbm.at[idx_v], rows_v, sem).wait()
        pltpu.sync_copy(rows_v, out_hbm.at[pl.ds(base, bpw)])

    return gk(table, idx)


def _pass1_lse(h, wt, b2):
    def k(h_ref, w_ref, b_ref, lse_ref, m_ref, s_ref):
        j = pl.program_id(0)

        @pl.when(j == 0)
        def _():
            m_ref[...] = jnp.full((BATCH, 1), -1e30, jnp.float32)
            s_ref[...] = jnp.zeros((BATCH, 1), jnp.float32)

        logits = (
            jnp.dot(h_ref[...], w_ref[...], preferred_element_type=jnp.float32)
            + b_ref[...]
        )
        col = lax.broadcasted_iota(jnp.int32, (1, TILE), 1) + j * TILE
        logits = jnp.where(col < VOCAB, logits, -1e30)
        m_old = m_ref[...]
        m_new = jnp.maximum(m_old, jnp.max(logits, axis=1, keepdims=True))
        s_ref[...] = s_ref[...] * jnp.exp(m_old - m_new) + jnp.sum(
            jnp.exp(logits - m_new), axis=1, keepdims=True
        )
        m_ref[...] = m_new

        @pl.when(j == NTILES - 1)
        def _():
            lse_ref[...] = m_ref[...] + jnp.log(s_ref[...])

    return pl.pallas_call(
        k,
        grid=(NTILES,),
        in_specs=[
            pl.BlockSpec((BATCH, EMBED), lambda j: (0, 0)),
            pl.BlockSpec((EMBED, TILE), lambda j: (0, j)),
            pl.BlockSpec((1, TILE), lambda j: (0, j)),
        ],
        out_specs=pl.BlockSpec((BATCH, 1), lambda j: (0, 0)),
        out_shape=jax.ShapeDtypeStruct((BATCH, 1), jnp.float32),
        scratch_shapes=[
            pltpu.VMEM((BATCH, 1), jnp.float32),
            pltpu.VMEM((BATCH, 1), jnp.float32),
        ],
    )(h, wt, b2)


def _pass2_out(h, wt, b2, lse):
    def k(h_ref, w_ref, b_ref, lse_ref, o_ref):
        logits = jnp.dot(h_ref[...], w_ref[...], preferred_element_type=jnp.float32)
        o_ref[...] = logits + (b_ref[...] - lse_ref[...])

    return pl.pallas_call(
        k,
        grid=(NTILES,),
        in_specs=[
            pl.BlockSpec((BATCH, EMBED), lambda j: (0, 0)),
            pl.BlockSpec((EMBED, TILE), lambda j: (0, j)),
            pl.BlockSpec((1, TILE), lambda j: (0, j)),
            pl.BlockSpec((BATCH, 1), lambda j: (0, 0)),
        ],
        out_specs=pl.BlockSpec((BATCH, TILE), lambda j: (0, j)),
        out_shape=jax.ShapeDtypeStruct((BATCH, VOCAB), jnp.float32),
    )(h, wt, b2, lse)


def kernel(x, table, W, b):
    h = _gather_sc(table, x.astype(jnp.int32))
    hb = h.astype(jnp.bfloat16)
    wt = W.T.astype(jnp.bfloat16)
    b2 = b.reshape(1, VOCAB)
    lse = _pass1_lse(hb, wt, b2)
    return _pass2_out(hb, wt, b2, lse)
